# baseline (speedup 1.0000x reference)
import jax
import jax.numpy as jnp
from jax.experimental import pallas as pl


def _copy_kernel(x_ref, o_ref):
    o_ref[...] = x_ref[...]


def _pna(x, src, dst, e, p, residual):
    f = jnp.concatenate([x[src], x[dst], e], axis=-1)
    msg = f @ p['Mw'].T + p['Mb']
    n = x.shape[0]
    ones = jnp.ones((src.shape[0],), jnp.float32)
    deg = jax.ops.segment_sum(ones, dst, num_segments=n)
    degc = jnp.maximum(deg, 1.0)[:, None]
    has = (deg > 0)[:, None]
    s = jax.ops.segment_sum(msg, dst, num_segments=n)
    mean = s / degc
    mx = jax.ops.segment_max(msg, dst, num_segments=n)
    mx = jnp.where(has, mx, 0.0)
    msq = jax.ops.segment_sum(msg * msg, dst, num_segments=n) / degc
    var = jax.nn.relu(msq - mean * mean)
    std = jnp.where(has, jnp.sqrt(var + 1e-30), 0.0)
    h_neigh = jnp.concatenate([mean, mx, s, std], axis=-1)
    h = jnp.concatenate([x, h_neigh], axis=-1) @ p['Uw'].T + p['Ub']
    mu = jnp.mean(h, axis=0)
    v = jnp.var(h, axis=0)
    h = (h - mu) / jnp.sqrt(v + 1e-5) * p['g'] + p['be']
    h = h @ p['mw'].T + p['mb']
    h = jnp.where(h >= 0, h, 0.01 * h)
    if residual:
        h = h + x
    return h


def kernel(n_feat, edge_index, e_feat, params):
    src = edge_index[0]
    dst = edge_index[1]
    perm = jnp.argsort(dst)
    src = src[perm]
    dst = dst[perm]
    e_feat = e_feat[perm]
    ns = n_feat[:, :-1]
    ns_s = n_feat[:, -1]
    h = jax.nn.relu(_pna(ns, src, dst, e_feat, params['L1'], True))
    h = jax.nn.relu(_pna(h, src, dst, e_feat, params['L2'], True))
    h = jax.nn.relu(_pna(h, src, dst, e_feat, params['L3'], False))
    t = jnp.concatenate([h, jnp.tile(ns_s[:, None], (1, 4))], axis=-1)
    t = jax.nn.relu(_pna(t, src, dst, e_feat, params['L4'], False))
    t = jax.nn.relu(_pna(t, src, dst, e_feat, params['L5'], True))
    t = _pna(t, src, dst, e_feat, params['L6'], False)
    soft = jax.nn.softmax(t, axis=0)
    out = soft[:, 0][None, :]
    out = pl.pallas_call(
        _copy_kernel,
        out_shape=jax.ShapeDtypeStruct(out.shape, out.dtype),
    )(out)
    return out


# speedup vs baseline: 1.3091x; 1.3091x over previous
import functools

import jax
import jax.numpy as jnp
from jax import lax
from jax.experimental import pallas as pl
from jax.experimental.pallas import tpu as pltpu

N_NODES = 50000
BN = 2000
NB = N_NODES // BN


def _pre_body(x_ref, w1_ref, w2_ref, mb_ref, a_ref, b_ref):
    x = x_ref[...]
    a_ref[...] = jnp.dot(x, w1_ref[...], preferred_element_type=jnp.float32) + mb_ref[...]
    b_ref[...] = jnp.dot(x, w2_ref[...], preferred_element_type=jnp.float32)


def _pre(x, w1, w2, mb):
    din = x.shape[1]
    dmid = w1.shape[1]
    return pl.pallas_call(
        _pre_body,
        grid=(NB,),
        in_specs=[
            pl.BlockSpec((BN, din), lambda i: (i, 0)),
            pl.BlockSpec((din, dmid), lambda i: (0, 0)),
            pl.BlockSpec((din, dmid), lambda i: (0, 0)),
            pl.BlockSpec((1, dmid), lambda i: (0, 0)),
        ],
        out_specs=[
            pl.BlockSpec((BN, dmid), lambda i: (i, 0)),
            pl.BlockSpec((BN, dmid), lambda i: (i, 0)),
        ],
        out_shape=[
            jax.ShapeDtypeStruct((N_NODES, dmid), jnp.float32),
            jax.ShapeDtypeStruct((N_NODES, dmid), jnp.float32),
        ],
    )(x, w1, w2, mb)


def _postA_body(x_ref, b_ref, s1_ref, s2_ref, mx_ref, degv_ref, uwt_ref, ub_ref,
                hpre_ref, stats_ref, acc_ref):
    i = pl.program_id(0)
    deg = degv_ref[:, :1]
    degc = jnp.maximum(deg, 1.0)
    has = deg > 0
    b = b_ref[...]
    s1 = s1_ref[...]
    s = s1 + deg * b
    mean = s / degc
    mx = jnp.where(has, mx_ref[...] + b, 0.0)
    msq = (s2_ref[...] + 2.0 * b * s1 + deg * b * b) / degc
    var = jax.nn.relu(msq - mean * mean)
    std = jnp.where(has, jnp.sqrt(var + 1e-30), 0.0)
    hn = jnp.concatenate([x_ref[...], mean, mx, s, std], axis=-1)
    h = jnp.dot(hn, uwt_ref[...], preferred_element_type=jnp.float32) + ub_ref[...]
    hpre_ref[...] = h

    @pl.when(i == 0)
    def _():
        acc_ref[...] = jnp.zeros_like(acc_ref)

    acc_ref[0:1, :] += jnp.sum(h, axis=0, keepdims=True)
    acc_ref[1:2, :] += jnp.sum(h * h, axis=0, keepdims=True)
    stats_ref[...] = acc_ref[...]


def _postA(x, b, s1, s2, mx, degv, uwt, ub):
    din = x.shape[1]
    dout = uwt.shape[1]
    dcat = uwt.shape[0]
    return pl.pallas_call(
        _postA_body,
        grid=(NB,),
        in_specs=[
            pl.BlockSpec((BN, din), lambda i: (i, 0)),
            pl.BlockSpec((BN, din), lambda i: (i, 0)),
            pl.BlockSpec((BN, din), lambda i: (i, 0)),
            pl.BlockSpec((BN, din), lambda i: (i, 0)),
            pl.BlockSpec((BN, din), lambda i: (i, 0)),
            pl.BlockSpec((BN, 16), lambda i: (i, 0)),
            pl.BlockSpec((dcat, dout), lambda i: (0, 0)),
            pl.BlockSpec((1, dout), lambda i: (0, 0)),
        ],
        out_specs=[
            pl.BlockSpec((BN, dout), lambda i: (i, 0)),
            pl.BlockSpec((8, dout), lambda i: (0, 0)),
        ],
        out_shape=[
            jax.ShapeDtypeStruct((N_NODES, dout), jnp.float32),
            jax.ShapeDtypeStruct((8, dout), jnp.float32),
        ],
        scratch_shapes=[pltpu.VMEM((8, dout), jnp.float32)],
    )(x, b, s1, s2, mx, degv, uwt, ub)


def _postB_body(hpre_ref, stats_ref, xres_ref, mwt_ref, mb_ref, g_ref, be_ref,
                out_ref, *, residual, final_relu):
    mu = stats_ref[0:1, :] / N_NODES
    var = stats_ref[1:2, :] / N_NODES - mu * mu
    inv = lax.rsqrt(var + 1e-5)
    h = (hpre_ref[...] - mu) * inv * g_ref[...] + be_ref[...]
    h = jnp.dot(h, mwt_ref[...], preferred_element_type=jnp.float32) + mb_ref[...]
    h = jnp.where(h >= 0, h, 0.01 * h)
    if residual:
        h = h + xres_ref[...]
    if final_relu:
        h = jax.nn.relu(h)
    out_ref[...] = h


def _postB(hpre, stats, xres, mwt, mb, g, be, residual, final_relu):
    dout = hpre.shape[1]
    body = functools.partial(_postB_body, residual=residual, final_relu=final_relu)
    return pl.pallas_call(
        body,
        grid=(NB,),
        in_specs=[
            pl.BlockSpec((BN, dout), lambda i: (i, 0)),
            pl.BlockSpec((8, dout), lambda i: (0, 0)),
            pl.BlockSpec((BN, dout), lambda i: (i, 0)),
            pl.BlockSpec((dout, dout), lambda i: (0, 0)),
            pl.BlockSpec((1, dout), lambda i: (0, 0)),
            pl.BlockSpec((1, dout), lambda i: (0, 0)),
            pl.BlockSpec((1, dout), lambda i: (0, 0)),
        ],
        out_specs=pl.BlockSpec((BN, dout), lambda i: (i, 0)),
        out_shape=jax.ShapeDtypeStruct((N_NODES, dout), jnp.float32),
    )(hpre, stats, xres, mwt, mb, g, be)


def _softmax_body(t_ref, out_ref):
    t = t_ref[...]
    m = jnp.max(t)
    ex = jnp.exp(t - m)
    out_ref[...] = ex / jnp.sum(ex)


def _softmax_row(trow):
    return pl.pallas_call(
        _softmax_body,
        out_shape=jax.ShapeDtypeStruct(trow.shape, jnp.float32),
    )(trow)


def _edge_stats_jax(a, src_s, dst_s, e0, e1, w3):
    m = a[src_s] + e0[:, None] * w3[0][None, :] + e1[:, None] * w3[1][None, :]
    s1 = jax.ops.segment_sum(m, dst_s, num_segments=N_NODES)
    s2 = jax.ops.segment_sum(m * m, dst_s, num_segments=N_NODES)
    mx = jax.ops.segment_max(m, dst_s, num_segments=N_NODES)
    mx = jnp.where((jax.ops.segment_sum(jnp.ones_like(e0), dst_s, num_segments=N_NODES) > 0)[:, None], mx, -1e30)
    return s1, s2, mx


def _layer(x, params, edge, degv, residual, final_relu):
    src_s, dst_s, e0, e1 = edge
    din = x.shape[1]
    p = params
    mw = p['Mw']
    dm = mw.shape[0]
    w1 = mw[:, :dm].T
    w2 = mw[:, dm:2 * dm].T
    w3 = mw[:, 2 * dm:].T
    mb2 = p['Mb'][None, :]
    if din != dm:
        w1p = jnp.zeros((din, din), jnp.float32).at[:dm, :dm].set(w1)
        w2p = jnp.zeros((din, din), jnp.float32).at[:dm, :dm].set(w2)
        w3p = jnp.zeros((2, din), jnp.float32).at[:, :dm].set(w3)
        mbp = jnp.zeros((1, din), jnp.float32).at[:, :dm].set(mb2)
        uwt = p['Uw'].T
        uwp = jnp.zeros((5 * din, uwt.shape[1]), jnp.float32)
        for blk in range(5):
            uwp = uwp.at[blk * din:blk * din + dm, :].set(uwt[blk * dm:(blk + 1) * dm, :])
        w1, w2, w3, mb2, uwt = w1p, w2p, w3p, mbp, uwp
    else:
        uwt = p['Uw'].T
    a, b = _pre(x, w1, w2, mb2)
    s1, s2, mx = _edge_stats_jax(a, src_s, dst_s, e0, e1, w3)
    hpre, stats = _postA(x, b, s1, s2, mx, degv, uwt, p['Ub'][None, :])
    xres = x if residual else hpre
    h = _postB(hpre, stats, xres, p['mw'].T, p['mb'][None, :], p['g'][None, :],
               p['be'][None, :], residual, final_relu)
    return h


def kernel(n_feat, edge_index, e_feat, params):
    src = edge_index[0]
    dst = edge_index[1]
    perm = jnp.argsort(dst)
    dst_s = dst[perm]
    src_s = src[perm]
    e_s = e_feat[perm]
    e0 = e_s[:, 0]
    e1 = e_s[:, 1]
    edge = (src_s, dst_s, e0, e1)

    deg = jax.ops.segment_sum(jnp.ones((src.shape[0],), jnp.float32), dst_s,
                              num_segments=N_NODES)
    degv = jnp.broadcast_to(deg[:, None], (N_NODES, 16))

    ns = n_feat[:, :-1]
    ns_s = n_feat[:, -1]
    h = _layer(ns, params['L1'], edge, degv, True, True)
    h = _layer(h, params['L2'], edge, degv, True, True)
    h = _layer(h, params['L3'], edge, degv, False, True)
    t = jnp.concatenate([h, jnp.tile(ns_s[:, None], (1, 4)),
                         jnp.zeros((N_NODES, 4), jnp.float32)], axis=-1)
    t = _layer(t, params['L4'], edge, degv, False, True)
    t = _layer(t, params['L5'], edge, degv, True, True)
    t = _layer(t, params['L6'], edge, degv, False, False)
    soft = _softmax_row(t[:, 0][None, :])
    return soft


# speedup vs baseline: 6.8752x; 5.2518x over previous
import functools

import jax
import jax.numpy as jnp
from jax import lax
from jax.experimental import pallas as pl
from jax.experimental.pallas import tpu as pltpu
from jax.experimental.pallas import tpu_sc as plsc

N_NODES = 50000
BN = 2000
NB = N_NODES // BN

NW = 32
SR = 256
KSUB = 224
KPW = KSUB // NW
EC = 128
E_EDGES = 800000
E_PAD = E_EDGES + 2 * EC


def _make_edge_kernel(dv, with_deg):
    d = dv * 16
    out_type = [jax.ShapeDtypeStruct((KSUB, SR, d), jnp.float32) for _ in range(3)]
    if with_deg:
        out_type.append(jax.ShapeDtypeStruct((KSUB, SR, 16), jnp.float32))
    scratch = [
        pltpu.VMEM((256,), jnp.int32),
        pltpu.VMEM((2 * d,), jnp.float32),
        pltpu.VMEM((2, EC), jnp.int32),
        pltpu.VMEM((2, 4 * EC + 16), jnp.int32),
        pltpu.VMEM((2, EC, d), jnp.float32),
        pltpu.VMEM((SR + 8, d), jnp.float32),
        pltpu.VMEM((SR + 8, d), jnp.float32),
        pltpu.VMEM((SR + 8, d), jnp.float32),
    ]
    if with_deg:
        scratch.append(pltpu.VMEM((SR + 8, 16), jnp.float32))
    scratch += [pltpu.SemaphoreType.DMA, pltpu.SemaphoreType.DMA]

    def body(a_hbm, src_hbm, pk_hbm, ss_hbm, w3_hbm, s1o, s2o, mxo, *rest):
        if with_deg:
            dgo = rest[0]
            (ss_v, w_v, src2, pk2, ar2,
             acc1, acc2, accm, accd, semL, semG) = rest[1:]
        else:
            (ss_v, w_v, src2, pk2, ar2,
             acc1, acc2, accm, semL, semG) = rest
        wid = lax.axis_index("s") * 2 + lax.axis_index("c")
        pltpu.sync_copy(ss_hbm, ss_v.at[pl.ds(0, 232)])
        pltpu.sync_copy(w3_hbm, w_v)
        w0 = [w_v[pl.ds(16 * j, 16)] for j in range(dv)]
        w1 = [w_v[pl.ds(d + 16 * j, 16)] for j in range(dv)]
        z16 = jnp.zeros((16,), jnp.float32)
        neg16 = jnp.full((16,), -1e30, jnp.float32)

        for kk in range(KPW):
            k_idx = wid * KPW + kk
            ssw = ss_v[pl.ds(k_idx, 16)]
            s0 = ssw[0]
            s1e = ssw[1]
            a0 = s0 - lax.rem(s0, 8)
            nch = lax.div(s1e - a0 + (EC - 1), EC)
            kbase = k_idx * SR

            def cbase(c):
                return pl.multiple_of(a0 + c * EC, 8)

            def start_lin(c):
                par = lax.rem(c, 2)
                base = cbase(c)
                pltpu.async_copy(src_hbm.at[pl.ds(base, EC)], src2.at[par], semL)
                pltpu.async_copy(pk_hbm.at[pl.ds(4 * base, 4 * EC)],
                                 pk2.at[par, pl.ds(0, 4 * EC)], semL)

            def wait_lin(c):
                par = lax.rem(c, 2)
                pltpu.make_async_copy(src_hbm.at[pl.ds(0, EC)],
                                      src2.at[par], semL).wait()
                pltpu.make_async_copy(pk_hbm.at[pl.ds(0, 4 * EC)],
                                      pk2.at[par, pl.ds(0, 4 * EC)], semL).wait()

            def start_gather(c):
                par = lax.rem(c, 2)
                pltpu.async_copy(a_hbm.at[src2.at[par]], ar2.at[par], semG)

            def wait_gather(c):
                par = lax.rem(c, 2)
                pltpu.make_async_copy(a_hbm.at[src2.at[par]],
                                      ar2.at[par], semG).wait()

            @pl.when(nch > 0)
            def _():
                start_lin(0)

            def zero_body(r, c):
                for j in range(dv):
                    acc1[r, pl.ds(16 * j, 16)] = z16
                    acc2[r, pl.ds(16 * j, 16)] = z16
                    accm[r, pl.ds(16 * j, 16)] = neg16
                if with_deg:
                    accd[r, pl.ds(0, 16)] = z16
                return c
            lax.fori_loop(0, SR + 8, zero_body, 0)

            @pl.when(nch > 0)
            def _():
                wait_lin(0)
                start_gather(0)

            @pl.when(nch > 1)
            def _():
                start_lin(1)

            def chunk_body(c, carry):
                wait_gather(c)

                @pl.when(c + 1 < nch)
                def _():
                    wait_lin(c + 1)
                    start_gather(c + 1)

                par = lax.rem(c, 2)
                base = cbase(c)

                def one_edge(i, carry):
                    pd, s1r, s2r, mxr, dgr = carry
                    li = pk2[par, pl.ds(4 * i, 16)]
                    lf = plsc.bitcast(li, jnp.float32)
                    dd = li[0]
                    ee0 = lf[1]
                    ee1 = lf[2]
                    idx = base + i
                    inr = jnp.logical_and(idx >= s0, idx < s1e)
                    dloc = jnp.where(inr, dd - kbase, SR)
                    chg = dloc != pd

                    @pl.when(chg)
                    def _():
                        for j in range(dv):
                            sl = pl.ds(16 * j, 16)
                            acc1[pd, sl] = s1r[j]
                            acc2[pd, sl] = s2r[j]
                            accm[pd, sl] = mxr[j]
                        if with_deg:
                            accd[pd, pl.ds(0, 16)] = dgr

                    ns1, ns2, nmx = [], [], []
                    for j in range(dv):
                        sl = pl.ds(16 * j, 16)
                        m = ar2[par, i, sl] + ee0 * w0[j] + ee1 * w1[j]
                        ns1.append(jnp.where(chg, z16, s1r[j]) + m)
                        ns2.append(jnp.where(chg, z16, s2r[j]) + m * m)
                        nmx.append(jnp.maximum(jnp.where(chg, neg16, mxr[j]), m))
                    ndg = dgr
                    if with_deg:
                        ndg = jnp.where(chg, z16, dgr) + 1.0
                    return (dloc, ns1, ns2, nmx, ndg)

                def edge_body(q, carry):
                    carry = one_edge(2 * q, carry)
                    return one_edge(2 * q + 1, carry)

                carry = lax.fori_loop(0, EC // 2, edge_body, carry)

                @pl.when(c + 2 < nch)
                def _():
                    start_lin(c + 2)
                return carry

            init = (jnp.int32(SR), [z16] * dv, [z16] * dv, [neg16] * dv, z16)
            pd, s1r, s2r, mxr, dgr = lax.fori_loop(0, nch, chunk_body, init)
            for j in range(dv):
                sl = pl.ds(16 * j, 16)
                acc1[pd, sl] = s1r[j]
                acc2[pd, sl] = s2r[j]
                accm[pd, sl] = mxr[j]
            if with_deg:
                accd[pd, pl.ds(0, 16)] = dgr

            pltpu.sync_copy(acc1.at[pl.ds(0, SR)], s1o.at[k_idx])
            pltpu.sync_copy(acc2.at[pl.ds(0, SR)], s2o.at[k_idx])
            pltpu.sync_copy(accm.at[pl.ds(0, SR)], mxo.at[k_idx])
            if with_deg:
                pltpu.sync_copy(accd.at[pl.ds(0, SR)], dgo.at[k_idx])

    mesh = plsc.VectorSubcoreMesh(core_axis_name="c", subcore_axis_name="s")
    return pl.kernel(body, mesh=mesh, out_type=out_type, scratch_types=scratch,
                     compiler_params=pltpu.CompilerParams(
                         use_tc_tiling_on_sc=False, needs_layout_passes=False))


_edge_k = {}
for _dv in (4, 1):
    for _wd in (True, False):
        _edge_k[(_dv, _wd)] = _make_edge_kernel(_dv, _wd)


def _pre_body(x_ref, w1_ref, w2_ref, mb_ref, a_ref, b_ref):
    x = x_ref[...]
    a_ref[...] = jnp.dot(x, w1_ref[...], preferred_element_type=jnp.float32) + mb_ref[...]
    b_ref[...] = jnp.dot(x, w2_ref[...], preferred_element_type=jnp.float32)


def _pre(x, w1, w2, mb):
    din = x.shape[1]
    dmid = w1.shape[1]
    return pl.pallas_call(
        _pre_body,
        grid=(NB,),
        in_specs=[
            pl.BlockSpec((BN, din), lambda i: (i, 0)),
            pl.BlockSpec((din, dmid), lambda i: (0, 0)),
            pl.BlockSpec((din, dmid), lambda i: (0, 0)),
            pl.BlockSpec((1, dmid), lambda i: (0, 0)),
        ],
        out_specs=[
            pl.BlockSpec((BN, dmid), lambda i: (i, 0)),
            pl.BlockSpec((BN, dmid), lambda i: (i, 0)),
        ],
        out_shape=[
            jax.ShapeDtypeStruct((N_NODES, dmid), jnp.float32),
            jax.ShapeDtypeStruct((N_NODES, dmid), jnp.float32),
        ],
    )(x, w1, w2, mb)


def _postA_body(x_ref, b_ref, s1_ref, s2_ref, mx_ref, degv_ref, uwt_ref, ub_ref,
                hpre_ref, stats_ref, acc_ref):
    i = pl.program_id(0)
    deg = degv_ref[:, :1]
    degc = jnp.maximum(deg, 1.0)
    has = deg > 0
    b = b_ref[...]
    s1 = s1_ref[...]
    s = s1 + deg * b
    mean = s / degc
    mx = jnp.where(has, mx_ref[...] + b, 0.0)
    msq = (s2_ref[...] + 2.0 * b * s1 + deg * b * b) / degc
    var = jax.nn.relu(msq - mean * mean)
    std = jnp.where(has, jnp.sqrt(var + 1e-30), 0.0)
    hn = jnp.concatenate([x_ref[...], mean, mx, s, std], axis=-1)
    h = jnp.dot(hn, uwt_ref[...], preferred_element_type=jnp.float32) + ub_ref[...]
    hpre_ref[...] = h

    @pl.when(i == 0)
    def _():
        acc_ref[...] = jnp.zeros_like(acc_ref)

    acc_ref[0:1, :] += jnp.sum(h, axis=0, keepdims=True)
    acc_ref[1:2, :] += jnp.sum(h * h, axis=0, keepdims=True)
    stats_ref[...] = acc_ref[...]


def _postA(x, b, s1, s2, mx, degv, uwt, ub):
    din = x.shape[1]
    dout = uwt.shape[1]
    dcat = uwt.shape[0]
    return pl.pallas_call(
        _postA_body,
        grid=(NB,),
        in_specs=[
            pl.BlockSpec((BN, din), lambda i: (i, 0)),
            pl.BlockSpec((BN, din), lambda i: (i, 0)),
            pl.BlockSpec((BN, din), lambda i: (i, 0)),
            pl.BlockSpec((BN, din), lambda i: (i, 0)),
            pl.BlockSpec((BN, din), lambda i: (i, 0)),
            pl.BlockSpec((BN, 16), lambda i: (i, 0)),
            pl.BlockSpec((dcat, dout), lambda i: (0, 0)),
            pl.BlockSpec((1, dout), lambda i: (0, 0)),
        ],
        out_specs=[
            pl.BlockSpec((BN, dout), lambda i: (i, 0)),
            pl.BlockSpec((8, dout), lambda i: (0, 0)),
        ],
        out_shape=[
            jax.ShapeDtypeStruct((N_NODES, dout), jnp.float32),
            jax.ShapeDtypeStruct((8, dout), jnp.float32),
        ],
        scratch_shapes=[pltpu.VMEM((8, dout), jnp.float32)],
    )(x, b, s1, s2, mx, degv, uwt, ub)


def _postB_body(hpre_ref, stats_ref, xres_ref, mwt_ref, mb_ref, g_ref, be_ref,
                out_ref, *, residual, final_relu):
    mu = stats_ref[0:1, :] / N_NODES
    var = stats_ref[1:2, :] / N_NODES - mu * mu
    inv = lax.rsqrt(var + 1e-5)
    h = (hpre_ref[...] - mu) * inv * g_ref[...] + be_ref[...]
    h = jnp.dot(h, mwt_ref[...], preferred_element_type=jnp.float32) + mb_ref[...]
    h = jnp.where(h >= 0, h, 0.01 * h)
    if residual:
        h = h + xres_ref[...]
    if final_relu:
        h = jax.nn.relu(h)
    out_ref[...] = h


def _postB(hpre, stats, xres, mwt, mb, g, be, residual, final_relu):
    dout = hpre.shape[1]
    body = functools.partial(_postB_body, residual=residual, final_relu=final_relu)
    return pl.pallas_call(
        body,
        grid=(NB,),
        in_specs=[
            pl.BlockSpec((BN, dout), lambda i: (i, 0)),
            pl.BlockSpec((8, dout), lambda i: (0, 0)),
            pl.BlockSpec((BN, dout), lambda i: (i, 0)),
            pl.BlockSpec((dout, dout), lambda i: (0, 0)),
            pl.BlockSpec((1, dout), lambda i: (0, 0)),
            pl.BlockSpec((1, dout), lambda i: (0, 0)),
            pl.BlockSpec((1, dout), lambda i: (0, 0)),
        ],
        out_specs=pl.BlockSpec((BN, dout), lambda i: (i, 0)),
        out_shape=jax.ShapeDtypeStruct((N_NODES, dout), jnp.float32),
    )(hpre, stats, xres, mwt, mb, g, be)


def _softmax_body(t_ref, out_ref):
    t = t_ref[...]
    m = jnp.max(t)
    ex = jnp.exp(t - m)
    out_ref[...] = ex / jnp.sum(ex)


def _softmax_row(trow):
    return pl.pallas_call(
        _softmax_body,
        out_shape=jax.ShapeDtypeStruct(trow.shape, jnp.float32),
    )(trow)


def _edge_stats(a, edge, w3, with_deg):
    src_p, pk, ss = edge
    d = a.shape[1]
    dv = d // 16
    outs = _edge_k[(dv, with_deg)](a, src_p, pk, ss, w3.reshape(-1))
    s1 = outs[0].reshape(KSUB * SR, d)
    s2 = outs[1].reshape(KSUB * SR, d)
    mx = outs[2].reshape(KSUB * SR, d)
    if with_deg:
        return s1, s2, mx, outs[3].reshape(KSUB * SR, 16)
    return s1, s2, mx


def _layer(x, params, edge, degv, residual, final_relu):
    din = x.shape[1]
    p = params
    mw = p['Mw']
    dm = mw.shape[0]
    w1 = mw[:, :dm].T
    w2 = mw[:, dm:2 * dm].T
    w3 = mw[:, 2 * dm:].T
    mb2 = p['Mb'][None, :]
    if din != dm:
        w1p = jnp.zeros((din, din), jnp.float32).at[:dm, :dm].set(w1)
        w2p = jnp.zeros((din, din), jnp.float32).at[:dm, :dm].set(w2)
        w3p = jnp.zeros((2, din), jnp.float32).at[:, :dm].set(w3)
        mbp = jnp.zeros((1, din), jnp.float32).at[:, :dm].set(mb2)
        uwt = p['Uw'].T
        uwp = jnp.zeros((5 * din, uwt.shape[1]), jnp.float32)
        for blk in range(5):
            uwp = uwp.at[blk * din:blk * din + dm, :].set(uwt[blk * dm:(blk + 1) * dm, :])
        w1, w2, w3, mb2, uwt = w1p, w2p, w3p, mbp, uwp
    else:
        uwt = p['Uw'].T
    a, b = _pre(x, w1, w2, mb2)
    if degv is None:
        s1, s2, mx, degv = _edge_stats(a, edge, w3, True)
    else:
        s1, s2, mx = _edge_stats(a, edge, w3, False)
    hpre, stats = _postA(x, b, s1, s2, mx, degv, uwt, p['Ub'][None, :])
    xres = x if residual else hpre
    h = _postB(hpre, stats, xres, p['mw'].T, p['mb'][None, :], p['g'][None, :],
               p['be'][None, :], residual, final_relu)
    return h, degv


def kernel(n_feat, edge_index, e_feat, params):
    src = edge_index[0]
    dst = edge_index[1]
    perm = jnp.argsort(dst)
    dst_s = dst[perm]
    src_s = src[perm]
    e_s = e_feat[perm]

    npad = E_PAD - E_EDGES
    src_p = jnp.concatenate([src_s, jnp.zeros((npad,), jnp.int32)])
    dst_p = jnp.concatenate([dst_s, jnp.full((npad,), N_NODES, jnp.int32)])
    e0_p = jnp.concatenate([e_s[:, 0], jnp.zeros((npad,), jnp.float32)])
    e1_p = jnp.concatenate([e_s[:, 1], jnp.zeros((npad,), jnp.float32)])
    pk = jnp.stack([dst_p,
                    lax.bitcast_convert_type(e0_p, jnp.int32),
                    lax.bitcast_convert_type(e1_p, jnp.int32),
                    jnp.zeros((E_PAD,), jnp.int32)], axis=1).reshape(-1)
    ss = jnp.searchsorted(dst_s, jnp.arange(KSUB + 1, dtype=jnp.int32) * SR)
    ss = jnp.concatenate([ss.astype(jnp.int32),
                          jnp.full((232 - (KSUB + 1),), E_EDGES, jnp.int32)])
    edge = (src_p, pk, ss)

    ns = n_feat[:, :-1]
    ns_s = n_feat[:, -1]
    h, degv = _layer(ns, params['L1'], edge, None, True, True)
    h, _ = _layer(h, params['L2'], edge, degv, True, True)
    h, _ = _layer(h, params['L3'], edge, degv, False, True)
    t = jnp.concatenate([h, jnp.tile(ns_s[:, None], (1, 4)),
                         jnp.zeros((N_NODES, 4), jnp.float32)], axis=-1)
    t, _ = _layer(t, params['L4'], edge, degv, False, True)
    t, _ = _layer(t, params['L5'], edge, degv, True, True)
    t, _ = _layer(t, params['L6'], edge, degv, False, False)
    soft = _softmax_row(t[:, 0][None, :])
    return soft
